# baseline (speedup 1.0000x reference)
import functools

import jax
import jax.numpy as jnp
from jax.experimental import pallas as pl
from jax.experimental.pallas import tpu as pltpu


def _gelu_exact(x):
    return 0.5 * x * (1.0 + jax.lax.erf(x * 0.7071067811865476))


def _expert_mlp_body(x_ref, wg_ref, wu_ref, wd_ref, w_ref, y_ref):
    j = pl.program_id(1)
    x = x_ref[0].astype(jnp.bfloat16)
    g = jax.lax.dot_general(x, wg_ref[0].astype(jnp.bfloat16),
                            (((1,), (1,)), ((), ())),
                            preferred_element_type=jnp.float32)
    u = jax.lax.dot_general(x, wu_ref[0].astype(jnp.bfloat16),
                            (((1,), (1,)), ((), ())),
                            preferred_element_type=jnp.float32)
    h = (_gelu_exact(g) * u).astype(jnp.bfloat16)
    y = jax.lax.dot_general(h, wd_ref[0].astype(jnp.bfloat16),
                            (((1,), (1,)), ((), ())),
                            preferred_element_type=jnp.float32)

    @pl.when(j == 0)
    def _init():
        y_ref[...] = jnp.zeros_like(y_ref)

    y_ref[0] += y

    @pl.when(j == pl.num_programs(1) - 1)
    def _scale():
        y_ref[0] = y_ref[0] * w_ref[0, 0][:, None]


def _expert_mlp(xg, exp_gate, exp_up, exp_down, topk_w, *, jb=256):
    E, C, d = xg.shape
    ff = exp_gate.shape[1]
    nj = ff // jb
    return pl.pallas_call(
        _expert_mlp_body,
        grid=(E, nj),
        in_specs=[
            pl.BlockSpec((1, C, d), lambda e, j: (e, 0, 0)),
            pl.BlockSpec((1, jb, d), lambda e, j: (e, j, 0)),
            pl.BlockSpec((1, jb, d), lambda e, j: (e, j, 0)),
            pl.BlockSpec((1, d, jb), lambda e, j: (e, 0, j)),
            pl.BlockSpec((1, 1, C), lambda e, j: (e, 0, 0)),
        ],
        out_specs=pl.BlockSpec((1, C, d), lambda e, j: (e, 0, 0)),
        out_shape=jax.ShapeDtypeStruct((E, C, d), jnp.float32),
        compiler_params=pltpu.CompilerParams(
            dimension_semantics=("parallel", "arbitrary")),
    )(xg, exp_gate, exp_up, exp_down, topk_w.reshape(E, 1, C))


def _shared_mlp_body(x_ref, g_ref, u_ref, d_ref, o_ref):
    x = x_ref[...].astype(jnp.bfloat16)
    g = jax.lax.dot_general(x, g_ref[...].astype(jnp.bfloat16),
                            (((1,), (1,)), ((), ())),
                            preferred_element_type=jnp.float32)
    u = jax.lax.dot_general(x, u_ref[...].astype(jnp.bfloat16),
                            (((1,), (1,)), ((), ())),
                            preferred_element_type=jnp.float32)
    h = (_gelu_exact(g) * u).astype(jnp.bfloat16)
    o_ref[...] = jax.lax.dot_general(h, d_ref[...].astype(jnp.bfloat16),
                                     (((1,), (1,)), ((), ())),
                                     preferred_element_type=jnp.float32)


def _shared_mlp(x, sh_gate, sh_up, sh_down, *, tb=512):
    N, d = x.shape
    sh = sh_gate.shape[0]
    nt = N // tb
    return pl.pallas_call(
        _shared_mlp_body,
        grid=(nt,),
        in_specs=[
            pl.BlockSpec((tb, d), lambda t: (t, 0)),
            pl.BlockSpec((sh, d), lambda t: (0, 0)),
            pl.BlockSpec((sh, d), lambda t: (0, 0)),
            pl.BlockSpec((d, sh), lambda t: (0, 0)),
        ],
        out_specs=pl.BlockSpec((tb, d), lambda t: (t, 0)),
        out_shape=jax.ShapeDtypeStruct((N, d), jnp.float32),
        compiler_params=pltpu.CompilerParams(
            dimension_semantics=("parallel",)),
    )(x, sh_gate, sh_up, sh_down)


def kernel(hidden_states, gate_w, exp_gate, exp_up, exp_down,
           sh_gate, sh_up, sh_down):
    B, S, d = hidden_states.shape
    E = gate_w.shape[0]
    N = B * S
    C = int(N * 2.0 / E)
    x = hidden_states.reshape(N, d)

    logits = x @ gate_w.T
    scores = jax.nn.softmax(logits, axis=-1)
    topk_w, topk_idx = jax.lax.top_k(scores.T, C)

    flat_idx = topk_idx.reshape(-1)
    xg = jnp.take(x, flat_idx, axis=0).reshape(E, C, d)
    y = _expert_mlp(xg, exp_gate, exp_up, exp_down, topk_w)

    out = _shared_mlp(x, sh_gate, sh_up, sh_down)
    out = out.at[flat_idx].add(y.reshape(N * 2, d))
    return out.reshape(B, S, d)


# speedup vs baseline: 1.3187x; 1.3187x over previous
import functools

import jax
import jax.numpy as jnp
from jax import lax
from jax.experimental import pallas as pl
from jax.experimental.pallas import tpu as pltpu
from jax.experimental.pallas import tpu_sc as plsc


def _gelu_exact(x):
    return 0.5 * x * (1.0 + jax.lax.erf(x * 0.7071067811865476))


def _expert_mlp_body(x_ref, wg_ref, wu_ref, wd_ref, w_ref, y_ref):
    j = pl.program_id(1)
    x = x_ref[0].astype(jnp.bfloat16)
    g = jax.lax.dot_general(x, wg_ref[0].astype(jnp.bfloat16),
                            (((1,), (1,)), ((), ())),
                            preferred_element_type=jnp.float32)
    u = jax.lax.dot_general(x, wu_ref[0].astype(jnp.bfloat16),
                            (((1,), (1,)), ((), ())),
                            preferred_element_type=jnp.float32)
    h = (_gelu_exact(g) * u).astype(jnp.bfloat16)
    y = jax.lax.dot_general(h, wd_ref[0].astype(jnp.bfloat16),
                            (((1,), (1,)), ((), ())),
                            preferred_element_type=jnp.float32)

    @pl.when(j == 0)
    def _init():
        y_ref[...] = jnp.zeros_like(y_ref)

    y_ref[0] += y

    @pl.when(j == pl.num_programs(1) - 1)
    def _scale():
        y_ref[0] = y_ref[0] * w_ref[0, 0][:, None]


def _expert_mlp(xg, exp_gate, exp_up, exp_down, topk_w, *, jb=512):
    E, C, d = xg.shape
    ff = exp_gate.shape[1]
    nj = ff // jb
    return pl.pallas_call(
        _expert_mlp_body,
        grid=(E, nj),
        in_specs=[
            pl.BlockSpec((1, C, d), lambda e, j: (e, 0, 0)),
            pl.BlockSpec((1, jb, d), lambda e, j: (e, j, 0)),
            pl.BlockSpec((1, jb, d), lambda e, j: (e, j, 0)),
            pl.BlockSpec((1, d, jb), lambda e, j: (e, 0, j)),
            pl.BlockSpec((1, 1, C), lambda e, j: (e, 0, 0)),
        ],
        out_specs=pl.BlockSpec((1, C, d), lambda e, j: (e, 0, 0)),
        out_shape=jax.ShapeDtypeStruct((E, C, d), jnp.float32),
        compiler_params=pltpu.CompilerParams(
            dimension_semantics=("parallel", "arbitrary")),
    )(xg, exp_gate, exp_up, exp_down, topk_w.reshape(E, 1, C))


def _shared_mlp_body(x_ref, g_ref, u_ref, d_ref, o_ref):
    x = x_ref[...].astype(jnp.bfloat16)
    g = jax.lax.dot_general(x, g_ref[...].astype(jnp.bfloat16),
                            (((1,), (1,)), ((), ())),
                            preferred_element_type=jnp.float32)
    u = jax.lax.dot_general(x, u_ref[...].astype(jnp.bfloat16),
                            (((1,), (1,)), ((), ())),
                            preferred_element_type=jnp.float32)
    h = (_gelu_exact(g) * u).astype(jnp.bfloat16)
    o_ref[...] = jax.lax.dot_general(h, d_ref[...].astype(jnp.bfloat16),
                                     (((1,), (1,)), ((), ())),
                                     preferred_element_type=jnp.float32)


def _shared_mlp(x, sh_gate, sh_up, sh_down, *, tb=512):
    N, d = x.shape
    sh = sh_gate.shape[0]
    nt = N // tb
    return pl.pallas_call(
        _shared_mlp_body,
        grid=(nt,),
        in_specs=[
            pl.BlockSpec((tb, d), lambda t: (t, 0)),
            pl.BlockSpec((sh, d), lambda t: (0, 0)),
            pl.BlockSpec((sh, d), lambda t: (0, 0)),
            pl.BlockSpec((d, sh), lambda t: (0, 0)),
        ],
        out_specs=pl.BlockSpec((tb, d), lambda t: (t, 0)),
        out_shape=jax.ShapeDtypeStruct((N, d), jnp.float32),
        compiler_params=pltpu.CompilerParams(
            dimension_semantics=("parallel",)),
    )(x, sh_gate, sh_up, sh_down)


def _routing_body(C, x_ref, g_ref, st_ref, thr_ref):
    lt = jax.lax.dot_general(g_ref[...], x_ref[...], (((1,), (1,)), ((), ())),
                             preferred_element_type=jnp.float32)
    m = jnp.max(lt, axis=0, keepdims=True)
    ex = jnp.exp(lt - m)
    sc = ex / jnp.sum(ex, axis=0, keepdims=True)
    st_ref[...] = sc
    bits = jax.lax.bitcast_convert_type(sc, jnp.int32)
    E = sc.shape[0]

    def body(_, carry):
        lo, hi = carry
        mid = jax.lax.div(lo + hi, 2)
        cnt = jnp.sum((bits >= mid).astype(jnp.int32), axis=1, keepdims=True)
        ge = cnt >= C
        return jnp.where(ge, mid, lo), jnp.where(ge, hi, mid)

    lo0 = jnp.zeros((E, 1), jnp.int32)
    hi0 = jnp.full((E, 1), 0x7F800000, jnp.int32)
    lo, _ = jax.lax.fori_loop(0, 31, body, (lo0, hi0))
    thr = jax.lax.bitcast_convert_type(lo, jnp.float32)
    thr_ref[...] = jnp.broadcast_to(thr, thr_ref.shape)


def _routing(x, gate_w, C):
    N, d = x.shape
    E = gate_w.shape[0]
    return pl.pallas_call(
        functools.partial(_routing_body, C),
        grid=(1,),
        in_specs=[pl.BlockSpec((N, d), lambda i: (0, 0)),
                  pl.BlockSpec((E, d), lambda i: (0, 0))],
        out_specs=[pl.BlockSpec((E, N), lambda i: (0, 0)),
                   pl.BlockSpec((E, 128), lambda i: (0, 0))],
        out_shape=[jax.ShapeDtypeStruct((E, N), jnp.float32),
                   jax.ShapeDtypeStruct((E, 128), jnp.float32)],
    )(x, gate_w)


def _compact_gather(scores_t, thr, x, C):
    E, N = scores_t.shape
    _, d = x.shape
    L = 16
    nv = N // L
    CH = 64
    mesh = plsc.VectorSubcoreMesh(core_axis_name="c", subcore_axis_name="s")

    @functools.partial(
        pl.kernel, mesh=mesh,
        compiler_params=pltpu.CompilerParams(needs_layout_passes=False),
        out_type=[jax.ShapeDtypeStruct((E, C), jnp.int32),
                  jax.ShapeDtypeStruct((E, C), jnp.float32),
                  jax.ShapeDtypeStruct((E * C, d), jnp.float32)],
        scratch_types=[pltpu.VMEM((N,), jnp.float32),
                       pltpu.VMEM((L,), jnp.float32),
                       pltpu.VMEM((C,), jnp.int32),
                       pltpu.VMEM((C,), jnp.float32),
                       pltpu.VMEM((CH, d), jnp.float32),
                       pltpu.SemaphoreType.DMA],
    )
    def k(scores_hbm, thr_hbm, x_hbm, idx_hbm, w_hbm, xg_hbm,
          s_v, t_v, idx_v, w_v, rows_v, sem):
        wid = lax.axis_index("s") * 2 + lax.axis_index("c")

        @pl.when(wid < E)
        def _():
            e = wid
            pltpu.sync_copy(scores_hbm.at[e], s_v)
            pltpu.sync_copy(thr_hbm.at[e, pl.ds(0, L)], t_v)
            t = t_v[...]

            def body(i, off):
                s = s_v[pl.ds(i * L, L)]
                msk = s >= t
                pos = off + plsc.cumsum(msk.astype(jnp.int32)) - 1
                m2 = jnp.logical_and(msk, pos < C)
                tok = lax.iota(jnp.int32, L) + i * L
                plsc.store_scatter(idx_v, [pos], tok, mask=m2)
                plsc.store_scatter(w_v, [pos], s, mask=m2)
                return off + plsc.all_reduce_population_count(msk)

            lax.fori_loop(0, nv, body, jnp.zeros((L,), jnp.int32))
            pltpu.sync_copy(idx_v, idx_hbm.at[e])
            pltpu.sync_copy(w_v, w_hbm.at[e])
            for c in range(C // CH):
                pltpu.async_copy(
                    x_hbm.at[idx_v.at[pl.ds(c * CH, CH)]], rows_v, sem).wait()
                pltpu.sync_copy(
                    rows_v, xg_hbm.at[pl.ds(e * C + c * CH, CH)])

    return k(scores_t, thr, x)


def kernel(hidden_states, gate_w, exp_gate, exp_up, exp_down,
           sh_gate, sh_up, sh_down):
    B, S, d = hidden_states.shape
    E = gate_w.shape[0]
    N = B * S
    C = int(N * 2.0 / E)
    x = hidden_states.reshape(N, d)

    scores_t, thr = _routing(x, gate_w, C)
    topk_idx, topk_w, xg_flat = _compact_gather(scores_t, thr, x, C)

    flat_idx = topk_idx.reshape(-1)
    y = _expert_mlp(xg_flat.reshape(E, C, d), exp_gate, exp_up, exp_down,
                    topk_w)

    out = _shared_mlp(x, sh_gate, sh_up, sh_down)
    out = out.at[flat_idx].add(y.reshape(N * 2, d))
    return out.reshape(B, S, d)
